# baseline (speedup 1.0000x reference)
import dataclasses
import functools

import jax
import jax.numpy as jnp
from jax import lax
from jax.experimental import pallas as pl
from jax.experimental.pallas import tpu as pltpu
from jax.experimental.pallas import tpu_sc as plsc

N = 10000
E = 320000
IN = 128
OUT = 128
R = 8
RN = R * N

NC = 2
NS = 16
NW = NC * NS
EPW = E // NW
B = 80
NB = EPW // B
L = 16
SEG = RN // NS

ROWS_A = 632
ROWS_B = N - 15 * ROWS_A


def _mesh():
    return plsc.VectorSubcoreMesh(core_axis_name="c", subcore_axis_name="s")


def _sc_params(**kw):
    cp = pltpu.CompilerParams()
    for name, val in ({"needs_layout_passes": False} | kw).items():
        if name in pltpu.CompilerParams.__dataclass_fields__:
            cp = dataclasses.replace(cp, **{name: val})
    return cp


def _zero_fill(ref, n):
    z = jnp.zeros((L,), jnp.float32)

    @pl.loop(0, n // L)
    def _(i):
        ref[pl.ds(i * L, L)] = z

    if n % L:
        ref[pl.ds(n - L, L)] = z


MASK14 = (1 << 14) - 1


def _unpack(p16):
    s16 = p16 & MASK14
    d16 = lax.shift_right_logical(p16, 14) & MASK14
    e16 = lax.shift_right_logical(p16, 28)
    return s16, d16, e16


def _sc_counts(packed):
    NR = 8

    @functools.partial(
        pl.kernel,
        mesh=_mesh(),
        out_type=jax.ShapeDtypeStruct((NC * RN,), jnp.float32),
        scratch_types=[
            pltpu.VMEM((EPW,), jnp.int32),
            [pltpu.VMEM((B,), jnp.int32) for _ in range(NR)],
            pltpu.VMEM((B,), jnp.float32),
            pltpu.VMEM((SEG,), jnp.float32),
            pltpu.VMEM_SHARED((RN,), jnp.float32),
            [pltpu.SemaphoreType.DMA for _ in range(NR)],
        ],
    )
    def k(pk_hbm, out_hbm, pkall, cidx, ones_v, stage, csh, sems):
        c = lax.axis_index("c")
        s = lax.axis_index("s")
        w = c * NS + s
        base = w * EPW
        _zero_fill(stage, SEG)
        pltpu.sync_copy(stage, csh.at[pl.ds(s * SEG, SEG)])
        pltpu.sync_copy(pk_hbm.at[pl.ds(base, EPW)], pkall)
        for t in range(B // L):
            ones_v[pl.ds(t * L, L)] = jnp.ones((L,), jnp.float32)
        plsc.subcore_barrier()

        def do_keys(j, u):
            o = j * B
            for t in range(B // L):
                _, d16, e16 = _unpack(pkall[pl.ds(o + t * L, L)])
                cidx[u][pl.ds(t * L, L)] = e16 * N + d16

        def start(u):
            pltpu.async_copy(ones_v, csh.at[cidx[u]], sems[u], add=True)

        def drain(u):
            pltpu.make_async_copy(ones_v, csh.at[cidx[u]], sems[u]).wait()

        for u in range(NR):
            do_keys(u, u)
            start(u)

        @pl.loop(1, NB // NR)
        def _(g):
            for u in range(NR):
                drain(u)
                do_keys(g * NR + u, u)
                start(u)

        for u in range(NB % NR):
            drain(u)
            do_keys((NB // NR) * NR + u, u)
            start(u)
        for u in range(NR):
            drain(u)

        plsc.subcore_barrier()
        pltpu.sync_copy(csh.at[pl.ds(s * SEG, SEG)], stage)
        pltpu.sync_copy(stage, out_hbm.at[pl.ds(c * RN + s * SEG, SEG)])

    return k(packed)


def _sc_agg(xw, packed, cnt):
    nset = 3

    @functools.partial(
        pl.kernel,
        mesh=_mesh(),
        out_type=jax.ShapeDtypeStruct((NC * N, OUT), jnp.float32),
        scratch_types=[
            pltpu.VMEM((EPW,), jnp.int32),
            [pltpu.VMEM((B,), jnp.int32) for _ in range(nset)],
            [pltpu.VMEM((B,), jnp.int32) for _ in range(nset)],
            [pltpu.VMEM((B,), jnp.int32) for _ in range(nset)],
            [pltpu.VMEM((B,), jnp.int32) for _ in range(nset)],
            [pltpu.VMEM((B,), jnp.float32) for _ in range(nset)],
            [pltpu.VMEM((B,), jnp.float32) for _ in range(nset)],
            [pltpu.VMEM((B,), jnp.float32) for _ in range(nset)],
            [pltpu.VMEM((B, OUT), jnp.float32) for _ in range(nset)],
            pltpu.VMEM_SHARED((N, OUT), jnp.float32),
            [pltpu.SemaphoreType.DMA for _ in range(nset)],
            [pltpu.SemaphoreType.DMA for _ in range(nset)],
            [pltpu.SemaphoreType.DMA for _ in range(nset)],
            [pltpu.SemaphoreType.DMA for _ in range(nset)],
        ],
        compiler_params=_sc_params(),
    )
    def k(xw_hbm, pk_hbm, cnt_hbm, out_hbm,
          pkall, gidx, cidxa, cidxb, didx, c0v, c1v, invv,
          rows, agg_sh, semr, sema, semb, semsc):
        c = lax.axis_index("c")
        s = lax.axis_index("s")
        w = c * NS + s
        base = w * EPW

        def do_idx(j, p):
            o = j * B
            for t in range(B // L):
                s16, d16, e16 = _unpack(pkall[pl.ds(o + t * L, L)])
                en = e16 * N
                ck = en + d16
                gidx[p][pl.ds(t * L, L)] = en + s16
                cidxa[p][pl.ds(t * L, L)] = ck
                cidxb[p][pl.ds(t * L, L)] = ck + RN
                didx[p][pl.ds(t * L, L)] = d16

        def fire(p):
            pltpu.make_async_copy(xw_hbm.at[gidx[p]], rows[p], semr[p]).start()
            pltpu.make_async_copy(cnt_hbm.at[cidxa[p]], c0v[p], sema[p]).start()
            pltpu.make_async_copy(cnt_hbm.at[cidxb[p]], c1v[p], semb[p]).start()

        def wait_inv(p):
            pltpu.make_async_copy(cnt_hbm.at[cidxa[p]], c0v[p], sema[p]).wait()
            pltpu.make_async_copy(cnt_hbm.at[cidxb[p]], c1v[p], semb[p]).wait()
            for t in range(B // L):
                sl = pl.ds(t * L, L)
                invv[p][sl] = 1.0 / (c0v[p][sl] + c1v[p][sl])
            pltpu.make_async_copy(xw_hbm.at[gidx[p]], rows[p], semr[p]).wait()

        def scale_scat(p):
            @pl.loop(0, B)
            def _(e):
                iv = plsc.load_gather(invv[p], [lax.broadcast(e, (L,))])
                for kk in range(OUT // L):
                    sl = (e, pl.ds(kk * L, L))
                    rows[p][sl] = rows[p][sl] * iv

            pltpu.async_copy(rows[p], agg_sh.at[didx[p]], semsc[p], add=True)

        def wait_scale(p):
            wait_inv(p)
            scale_scat(p)

        def wait_scat(p):
            pltpu.make_async_copy(rows[p], agg_sh.at[didx[p]], semsc[p]).wait()

        @pl.loop(0, B)
        def _(e):
            for kk in range(OUT // L):
                rows[0][e, pl.ds(kk * L, L)] = jnp.zeros((L,), jnp.float32)

        @pl.when(s < NS - 1)
        def _():
            rb = s * ROWS_A

            @pl.loop(0, ROWS_A // B)
            def _(j):
                pltpu.sync_copy(rows[0], agg_sh.at[pl.ds(rb + j * B, B)])

            rem = ROWS_A % B
            pltpu.sync_copy(rows[0].at[pl.ds(0, rem)],
                            agg_sh.at[pl.ds(rb + ROWS_A - rem, rem)])

        @pl.when(s == NS - 1)
        def _():
            rb = (NS - 1) * ROWS_A

            @pl.loop(0, ROWS_B // B)
            def _(j):
                pltpu.sync_copy(rows[0], agg_sh.at[pl.ds(rb + j * B, B)])

            rem = ROWS_B % B
            pltpu.sync_copy(rows[0].at[pl.ds(0, rem)],
                            agg_sh.at[pl.ds(rb + ROWS_B - rem, rem)])

        pltpu.sync_copy(pk_hbm.at[pl.ds(base, EPW)], pkall)
        plsc.subcore_barrier()

        do_idx(0, 0)
        fire(0)
        do_idx(1, 1)
        fire(1)

        wait_scale(0)
        do_idx(2, 2)
        fire(2)
        wait_scale(1)
        wait_scat(0)
        do_idx(3, 0)
        fire(0)
        wait_scale(2)
        wait_scat(1)
        do_idx(4, 1)
        fire(1)

        @pl.loop(0, (NB - 5) // 3)
        def _(t):
            j = 3 * t + 3
            for o, (p, pn) in enumerate(((0, 2), (1, 0), (2, 1))):
                wait_scale(p)
                wait_scat(pn)
                do_idx(j + o + 2, pn)
                fire(pn)

        wait_scale(0)
        wait_scale(1)
        wait_scat(2)
        wait_scat(0)
        wait_scat(1)
        plsc.subcore_barrier()

        @pl.when(s < NS - 1)
        def _():
            rb = s * ROWS_A

            @pl.loop(0, ROWS_A // B)
            def _(j):
                pltpu.sync_copy(agg_sh.at[pl.ds(rb + j * B, B)], rows[0])
                pltpu.sync_copy(rows[0],
                                out_hbm.at[pl.ds(c * N + rb + j * B, B)])

            rem = ROWS_A % B
            pltpu.sync_copy(agg_sh.at[pl.ds(rb + ROWS_A - rem, rem)],
                            rows[0].at[pl.ds(0, rem)])
            pltpu.sync_copy(rows[0].at[pl.ds(0, rem)],
                            out_hbm.at[pl.ds(c * N + rb + ROWS_A - rem, rem)])

        @pl.when(s == NS - 1)
        def _():
            rb = (NS - 1) * ROWS_A

            @pl.loop(0, ROWS_B // B)
            def _(j):
                pltpu.sync_copy(agg_sh.at[pl.ds(rb + j * B, B)], rows[0])
                pltpu.sync_copy(rows[0],
                                out_hbm.at[pl.ds(c * N + rb + j * B, B)])

            rem = ROWS_B % B
            pltpu.sync_copy(agg_sh.at[pl.ds(rb + ROWS_B - rem, rem)],
                            rows[0].at[pl.ds(0, rem)])
            pltpu.sync_copy(rows[0].at[pl.ds(0, rem)],
                            out_hbm.at[pl.ds(c * N + rb + ROWS_B - rem, rem)])

    return k(xw, packed, cnt)


def _tc_xw(x, W_rel):
    BN = 2000

    def body(x_ref, w_ref, o_ref):
        o_ref[0] = jnp.dot(x_ref[...], w_ref[0],
                           preferred_element_type=jnp.float32)

    return pl.pallas_call(
        body,
        grid=(R, N // BN),
        in_specs=[
            pl.BlockSpec((BN, IN), lambda r, i: (i, 0)),
            pl.BlockSpec((1, IN, OUT), lambda r, i: (r, 0, 0)),
        ],
        out_specs=pl.BlockSpec((1, BN, OUT), lambda r, i: (r, i, 0)),
        out_shape=jax.ShapeDtypeStruct((R, N, OUT), jnp.float32),
    )(x, W_rel)


def _tc_final(x, agg0, agg1, W_root, b_root2, Wgu, Wgx, bg2):
    BN = 2000

    def body(x_ref, a0_ref, a1_ref, wr_ref, br_ref, wu_ref, wx_ref, bg_ref,
             o_ref):
        xb = x_ref[...]
        u = (jnp.dot(xb, wr_ref[...], preferred_element_type=jnp.float32)
             + br_ref[...] + a0_ref[...] + a1_ref[...])
        z = (jnp.dot(u, wu_ref[...], preferred_element_type=jnp.float32)
             + jnp.dot(xb, wx_ref[...], preferred_element_type=jnp.float32)
             + bg_ref[...])
        o_ref[...] = jnp.tanh(u) * z + xb * (1.0 - z)

    return pl.pallas_call(
        body,
        grid=(N // BN,),
        in_specs=[
            pl.BlockSpec((BN, IN), lambda i: (i, 0)),
            pl.BlockSpec((BN, OUT), lambda i: (i, 0)),
            pl.BlockSpec((BN, OUT), lambda i: (i, 0)),
            pl.BlockSpec((IN, OUT), lambda i: (0, 0)),
            pl.BlockSpec((1, OUT), lambda i: (0, 0)),
            pl.BlockSpec((OUT, OUT), lambda i: (0, 0)),
            pl.BlockSpec((IN, OUT), lambda i: (0, 0)),
            pl.BlockSpec((1, OUT), lambda i: (0, 0)),
        ],
        out_specs=pl.BlockSpec((BN, OUT), lambda i: (i, 0)),
        out_shape=jax.ShapeDtypeStruct((N, OUT), jnp.float32),
    )(x, agg0, agg1, W_root, b_root2, Wgu, Wgx, bg2)


def kernel(x, edge_index, edge_type, W_rel, W_root, b_root, Wg, bg):
    src = edge_index[0]
    dst = edge_index[1]
    packed = src | (dst << 14) | (edge_type << 28)

    xw = _tc_xw(x, W_rel).reshape(RN, OUT)
    cnt = _sc_counts(packed)
    ap = _sc_agg(xw, packed, cnt).reshape(NC, N, OUT)
    return _tc_final(x, ap[0], ap[1], W_root, b_root.reshape(1, OUT),
                     Wg[:IN], Wg[IN:], bg.reshape(1, OUT))


# speedup vs baseline: 1.0038x; 1.0038x over previous
import dataclasses
import functools

import jax
import jax.numpy as jnp
from jax import lax
from jax.experimental import pallas as pl
from jax.experimental.pallas import tpu as pltpu
from jax.experimental.pallas import tpu_sc as plsc

N = 10000
E = 320000
IN = 128
OUT = 128
R = 8
RN = R * N

NC = 2
NS = 16
NW = NC * NS
EPW = E // NW
B = 80
NB = EPW // B
L = 16
SEG = RN // NS

ROWS_A = 632
ROWS_B = N - 15 * ROWS_A


def _mesh():
    return plsc.VectorSubcoreMesh(core_axis_name="c", subcore_axis_name="s")


def _sc_params(**kw):
    cp = pltpu.CompilerParams()
    for name, val in ({"needs_layout_passes": False} | kw).items():
        if name in pltpu.CompilerParams.__dataclass_fields__:
            cp = dataclasses.replace(cp, **{name: val})
    return cp


def _zero_fill(ref, n):
    z = jnp.zeros((L,), jnp.float32)

    @pl.loop(0, n // L)
    def _(i):
        ref[pl.ds(i * L, L)] = z

    if n % L:
        ref[pl.ds(n - L, L)] = z


MASK14 = (1 << 14) - 1


def _unpack(p16):
    s16 = p16 & MASK14
    d16 = lax.shift_right_logical(p16, 14) & MASK14
    e16 = lax.shift_right_logical(p16, 28)
    return s16, d16, e16


def _sc_counts(packed):
    @functools.partial(
        pl.kernel,
        mesh=_mesh(),
        out_type=jax.ShapeDtypeStruct((NC * RN,), jnp.float32),
        scratch_types=[
            pltpu.VMEM((EPW,), jnp.int32),
            pltpu.VMEM((B,), jnp.int32),
            pltpu.VMEM((B,), jnp.float32),
            pltpu.VMEM((SEG,), jnp.float32),
            pltpu.VMEM_SHARED((RN,), jnp.float32),
        ],
    )
    def k(pk_hbm, out_hbm, pkall, cidx, ones_v, stage, csh):
        c = lax.axis_index("c")
        s = lax.axis_index("s")
        w = c * NS + s
        base = w * EPW
        _zero_fill(stage, SEG)
        pltpu.sync_copy(stage, csh.at[pl.ds(s * SEG, SEG)])
        pltpu.sync_copy(pk_hbm.at[pl.ds(base, EPW)], pkall)
        for t in range(B // L):
            ones_v[pl.ds(t * L, L)] = jnp.ones((L,), jnp.float32)
        plsc.subcore_barrier()

        @pl.loop(0, NB)
        def _(j):
            o = j * B
            for t in range(B // L):
                _, d16, e16 = _unpack(pkall[pl.ds(o + t * L, L)])
                cidx[pl.ds(t * L, L)] = e16 * N + d16
            pltpu.sync_copy(ones_v, csh.at[cidx], add=True)

        plsc.subcore_barrier()
        pltpu.sync_copy(csh.at[pl.ds(s * SEG, SEG)], stage)
        pltpu.sync_copy(stage, out_hbm.at[pl.ds(c * RN + s * SEG, SEG)])

    return k(packed)


def _sc_agg(xw, packed, cnt):
    nset = 3

    @functools.partial(
        pl.kernel,
        mesh=_mesh(),
        out_type=jax.ShapeDtypeStruct((NC * N, OUT), jnp.float32),
        scratch_types=[
            pltpu.VMEM((EPW,), jnp.int32),
            [pltpu.VMEM((B,), jnp.int32) for _ in range(nset)],
            [pltpu.VMEM((B,), jnp.int32) for _ in range(nset)],
            [pltpu.VMEM((B,), jnp.int32) for _ in range(nset)],
            [pltpu.VMEM((B,), jnp.int32) for _ in range(nset)],
            [pltpu.VMEM((B,), jnp.float32) for _ in range(nset)],
            [pltpu.VMEM((B,), jnp.float32) for _ in range(nset)],
            [pltpu.VMEM((B,), jnp.float32) for _ in range(nset)],
            [pltpu.VMEM((B, OUT), jnp.float32) for _ in range(nset)],
            pltpu.VMEM_SHARED((N, OUT), jnp.float32),
            [pltpu.SemaphoreType.DMA for _ in range(nset)],
            [pltpu.SemaphoreType.DMA for _ in range(nset)],
            [pltpu.SemaphoreType.DMA for _ in range(nset)],
            [pltpu.SemaphoreType.DMA for _ in range(nset)],
        ],
        compiler_params=_sc_params(),
    )
    def k(xw_hbm, pk_hbm, cnt_hbm, out_hbm,
          pkall, gidx, cidxa, cidxb, didx, c0v, c1v, invv,
          rows, agg_sh, semr, sema, semb, semsc):
        c = lax.axis_index("c")
        s = lax.axis_index("s")
        w = c * NS + s
        base = w * EPW

        def do_idx(j, p):
            o = j * B
            for t in range(B // L):
                s16, d16, e16 = _unpack(pkall[pl.ds(o + t * L, L)])
                en = e16 * N
                ck = en + d16
                gidx[p][pl.ds(t * L, L)] = en + s16
                cidxa[p][pl.ds(t * L, L)] = ck
                cidxb[p][pl.ds(t * L, L)] = ck + RN
                didx[p][pl.ds(t * L, L)] = d16

        def fire(p):
            pltpu.make_async_copy(xw_hbm.at[gidx[p]], rows[p], semr[p]).start()
            pltpu.make_async_copy(cnt_hbm.at[cidxa[p]], c0v[p], sema[p]).start()
            pltpu.make_async_copy(cnt_hbm.at[cidxb[p]], c1v[p], semb[p]).start()

        def wait_inv(p):
            pltpu.make_async_copy(cnt_hbm.at[cidxa[p]], c0v[p], sema[p]).wait()
            pltpu.make_async_copy(cnt_hbm.at[cidxb[p]], c1v[p], semb[p]).wait()
            for t in range(B // L):
                sl = pl.ds(t * L, L)
                invv[p][sl] = 1.0 / (c0v[p][sl] + c1v[p][sl])
            pltpu.make_async_copy(xw_hbm.at[gidx[p]], rows[p], semr[p]).wait()

        def scale_scat(p):
            @pl.loop(0, B)
            def _(e):
                iv = plsc.load_gather(invv[p], [lax.broadcast(e, (L,))])
                for kk in range(OUT // L):
                    sl = (e, pl.ds(kk * L, L))
                    rows[p][sl] = rows[p][sl] * iv

            pltpu.async_copy(rows[p], agg_sh.at[didx[p]], semsc[p], add=True)

        def wait_scale(p):
            wait_inv(p)
            scale_scat(p)

        def wait_scat(p):
            pltpu.make_async_copy(rows[p], agg_sh.at[didx[p]], semsc[p]).wait()

        @pl.loop(0, B)
        def _(e):
            for kk in range(OUT // L):
                rows[0][e, pl.ds(kk * L, L)] = jnp.zeros((L,), jnp.float32)

        @pl.when(s < NS - 1)
        def _():
            rb = s * ROWS_A

            @pl.loop(0, ROWS_A // B)
            def _(j):
                pltpu.sync_copy(rows[0], agg_sh.at[pl.ds(rb + j * B, B)])

            rem = ROWS_A % B
            pltpu.sync_copy(rows[0].at[pl.ds(0, rem)],
                            agg_sh.at[pl.ds(rb + ROWS_A - rem, rem)])

        @pl.when(s == NS - 1)
        def _():
            rb = (NS - 1) * ROWS_A

            @pl.loop(0, ROWS_B // B)
            def _(j):
                pltpu.sync_copy(rows[0], agg_sh.at[pl.ds(rb + j * B, B)])

            rem = ROWS_B % B
            pltpu.sync_copy(rows[0].at[pl.ds(0, rem)],
                            agg_sh.at[pl.ds(rb + ROWS_B - rem, rem)])

        pltpu.sync_copy(pk_hbm.at[pl.ds(base, EPW)], pkall)
        plsc.subcore_barrier()

        do_idx(0, 0)
        fire(0)
        do_idx(1, 1)
        fire(1)

        wait_scale(0)
        do_idx(2, 2)
        fire(2)
        wait_scale(1)
        wait_scat(0)
        do_idx(3, 0)
        fire(0)
        wait_scale(2)
        wait_scat(1)
        do_idx(4, 1)
        fire(1)

        @pl.loop(0, (NB - 5) // 3)
        def _(t):
            j = 3 * t + 3
            for o, (p, pn) in enumerate(((0, 2), (1, 0), (2, 1))):
                wait_scale(p)
                wait_scat(pn)
                do_idx(j + o + 2, pn)
                fire(pn)

        wait_scale(0)
        wait_scale(1)
        wait_scat(2)
        wait_scat(0)
        wait_scat(1)
        plsc.subcore_barrier()

        @pl.when(s < NS - 1)
        def _():
            rb = s * ROWS_A

            @pl.loop(0, ROWS_A // B)
            def _(j):
                pltpu.sync_copy(agg_sh.at[pl.ds(rb + j * B, B)], rows[0])
                pltpu.sync_copy(rows[0],
                                out_hbm.at[pl.ds(c * N + rb + j * B, B)])

            rem = ROWS_A % B
            pltpu.sync_copy(agg_sh.at[pl.ds(rb + ROWS_A - rem, rem)],
                            rows[0].at[pl.ds(0, rem)])
            pltpu.sync_copy(rows[0].at[pl.ds(0, rem)],
                            out_hbm.at[pl.ds(c * N + rb + ROWS_A - rem, rem)])

        @pl.when(s == NS - 1)
        def _():
            rb = (NS - 1) * ROWS_A

            @pl.loop(0, ROWS_B // B)
            def _(j):
                pltpu.sync_copy(agg_sh.at[pl.ds(rb + j * B, B)], rows[0])
                pltpu.sync_copy(rows[0],
                                out_hbm.at[pl.ds(c * N + rb + j * B, B)])

            rem = ROWS_B % B
            pltpu.sync_copy(agg_sh.at[pl.ds(rb + ROWS_B - rem, rem)],
                            rows[0].at[pl.ds(0, rem)])
            pltpu.sync_copy(rows[0].at[pl.ds(0, rem)],
                            out_hbm.at[pl.ds(c * N + rb + ROWS_B - rem, rem)])

    return k(xw, packed, cnt)


def _tc_xw(x, W_rel):
    BN = 2000

    def body(x_ref, w_ref, o_ref):
        o_ref[0] = jnp.dot(x_ref[...], w_ref[0],
                           preferred_element_type=jnp.float32)

    return pl.pallas_call(
        body,
        grid=(R, N // BN),
        in_specs=[
            pl.BlockSpec((BN, IN), lambda r, i: (i, 0)),
            pl.BlockSpec((1, IN, OUT), lambda r, i: (r, 0, 0)),
        ],
        out_specs=pl.BlockSpec((1, BN, OUT), lambda r, i: (r, i, 0)),
        out_shape=jax.ShapeDtypeStruct((R, N, OUT), jnp.float32),
    )(x, W_rel)


def _tc_final(x, agg0, agg1, W_root, b_root2, Wgu, Wgx, bg2):
    BN = 2000

    def body(x_ref, a0_ref, a1_ref, wr_ref, br_ref, wu_ref, wx_ref, bg_ref,
             o_ref):
        xb = x_ref[...]
        u = (jnp.dot(xb, wr_ref[...], preferred_element_type=jnp.float32)
             + br_ref[...] + a0_ref[...] + a1_ref[...])
        z = (jnp.dot(u, wu_ref[...], preferred_element_type=jnp.float32)
             + jnp.dot(xb, wx_ref[...], preferred_element_type=jnp.float32)
             + bg_ref[...])
        o_ref[...] = jnp.tanh(u) * z + xb * (1.0 - z)

    return pl.pallas_call(
        body,
        grid=(N // BN,),
        in_specs=[
            pl.BlockSpec((BN, IN), lambda i: (i, 0)),
            pl.BlockSpec((BN, OUT), lambda i: (i, 0)),
            pl.BlockSpec((BN, OUT), lambda i: (i, 0)),
            pl.BlockSpec((IN, OUT), lambda i: (0, 0)),
            pl.BlockSpec((1, OUT), lambda i: (0, 0)),
            pl.BlockSpec((OUT, OUT), lambda i: (0, 0)),
            pl.BlockSpec((IN, OUT), lambda i: (0, 0)),
            pl.BlockSpec((1, OUT), lambda i: (0, 0)),
        ],
        out_specs=pl.BlockSpec((BN, OUT), lambda i: (i, 0)),
        out_shape=jax.ShapeDtypeStruct((N, OUT), jnp.float32),
    )(x, agg0, agg1, W_root, b_root2, Wgu, Wgx, bg2)


def kernel(x, edge_index, edge_type, W_rel, W_root, b_root, Wg, bg):
    src = edge_index[0]
    dst = edge_index[1]
    packed = src | (dst << 14) | (edge_type << 28)

    xw = _tc_xw(x, W_rel).reshape(RN, OUT)
    cnt = _sc_counts(packed)
    ap = _sc_agg(xw, packed, cnt).reshape(NC, N, OUT)
    return _tc_final(x, ap[0], ap[1], W_root, b_root.reshape(1, OUT),
                     Wg[:IN], Wg[IN:], bg.reshape(1, OUT))
